# Initial kernel scaffold; baseline (speedup 1.0000x reference)
import jax
import jax.numpy as jnp
from jax.experimental import pallas as pl


def kernel(x, embeddings):
    raise NotImplementedError("write your pallas kernel here")


# speedup vs baseline: 1.3953x; 1.3953x over previous
import functools

import jax
import jax.numpy as jnp
from jax import lax
from jax.experimental import pallas as pl
from jax.experimental.pallas import tpu as pltpu
from jax.experimental.pallas import tpu_sc as plsc

D = 256
K = 8192
N = 8192
RT = 256
CT = 1024
BETA = 0.25


def _k1_body(x_ref, e_ref, idx_ref, loss_ref, e2_ref, acc_ref):
    i = pl.program_id(0)

    @pl.when(i == 0)
    def _():
        e = e_ref[...]
        e2_ref[...] = jnp.sum(e * e, axis=0, keepdims=True)
        acc_ref[0, 0] = 0.0

    xb = x_ref[...]
    x2 = jnp.sum(xb * xb, axis=1, keepdims=True)
    m = lax.dot_general(xb, e_ref[...], (((1,), (0,)), ((), ())),
                        preferred_element_type=jnp.float32)
    d = (x2 - 2.0 * m) + e2_ref[...]
    dmin = jnp.min(d, axis=1, keepdims=True)
    acc_ref[0, 0] += jnp.sum(dmin)
    ids = lax.broadcasted_iota(jnp.int32, (RT, K), 1)
    cand = jnp.where(d == dmin, ids, jnp.int32(2147483647))
    idx_ref[...] = jnp.min(cand, axis=1, keepdims=True)

    @pl.when(i == pl.num_programs(0) - 1)
    def _():
        loss_ref[...] = jnp.full((1, 1), (1.0 + BETA) / (N * D),
                                 jnp.float32) * acc_ref[0, 0]


def _k1(xf, emb):
    return pl.pallas_call(
        _k1_body,
        grid=(N // RT,),
        in_specs=[pl.BlockSpec((RT, D), lambda i: (i, 0)),
                  pl.BlockSpec((D, K), lambda i: (0, 0))],
        out_specs=[pl.BlockSpec((RT, 1), lambda i: (i, 0)),
                   pl.BlockSpec((1, 1), lambda i: (0, 0))],
        out_shape=[jax.ShapeDtypeStruct((N, 1), jnp.int32),
                   jax.ShapeDtypeStruct((1, 1), jnp.float32)],
        scratch_shapes=[pltpu.VMEM((1, K), jnp.float32),
                        pltpu.SMEM((1, 1), jnp.float32)],
    )(xf, emb)


def _k3_body(idx_ref, out_ref, cnt_ref):
    i = pl.program_id(0)

    @pl.when(i == 0)
    def _():
        cnt_ref[...] = jnp.zeros_like(cnt_ref)

    idxb = idx_ref[...]
    ids = lax.broadcasted_iota(jnp.int32, (CT, K), 1)
    eq = (idxb == ids).astype(jnp.float32)
    cnt_ref[...] += jnp.sum(eq, axis=0, keepdims=True)

    @pl.when(i == pl.num_programs(0) - 1)
    def _():
        p = cnt_ref[...] * (1.0 / N)
        h = -jnp.sum(p * jnp.log(p + 1e-10))
        out_ref[...] = jnp.exp(h) * jnp.ones((1, 1), jnp.float32)


def _k3(idx2):
    return pl.pallas_call(
        _k3_body,
        grid=(N // CT,),
        in_specs=[pl.BlockSpec((CT, 1), lambda i: (i, 0))],
        out_specs=pl.BlockSpec((1, 1), lambda i: (0, 0)),
        out_shape=jax.ShapeDtypeStruct((1, 1), jnp.float32),
        scratch_shapes=[pltpu.VMEM((1, K), jnp.float32)],
    )(idx2)


_NW = 32
_BPW = N // _NW


@functools.cache
def _k2_build():
    @functools.partial(
        pl.kernel,
        mesh=plsc.VectorSubcoreMesh(core_axis_name="c", subcore_axis_name="s"),
        out_type=jax.ShapeDtypeStruct((N, D), jnp.float32),
        scratch_types=[pltpu.VMEM((_BPW,), jnp.int32),
                       pltpu.VMEM((_BPW, D), jnp.float32),
                       pltpu.SemaphoreType.DMA],
    )
    def _k2(table_hbm, idx_hbm, out_hbm, idx_v, rows_v, sem):
        wid = lax.axis_index("s") * 2 + lax.axis_index("c")
        base = wid * _BPW
        pltpu.sync_copy(idx_hbm.at[pl.ds(base, _BPW)], idx_v)
        pltpu.async_copy(table_hbm.at[idx_v], rows_v, sem).wait()
        pltpu.sync_copy(rows_v, out_hbm.at[pl.ds(base, _BPW)])

    return _k2


def kernel(x, embeddings):
    xf = x.reshape(N, D)
    idx2, loss = _k1(xf, embeddings)
    perp = _k3(idx2)
    quant = _k2_build()(embeddings.T, idx2.reshape(N))
    return (quant.reshape(x.shape), loss.reshape(()), perp.reshape(()),
            idx2.reshape(x.shape[:-1]))
